# baseline (speedup 1.0000x reference)
import functools

import jax
import jax.numpy as jnp
from jax import lax
from jax.experimental import pallas as pl
from jax.experimental.pallas import tpu as pltpu
from jax.experimental.pallas import tpu_sc as plsc

VOCAB = 1000000
EMB = 64
HID = 1000
NCLS = 1000
B = 16384
L = 200

NC, NS = 2, 16
NW = NC * NS
CB = 4
GSZ = 80
IPC = CB * L
NG = IPC // GSZ
NCHUNK = B // CB
CPW = NCHUNK // NW

HIDP = 1024
NCLSP = 1024
BM = 1024


def _pool_body(idx_hbm, table_hbm, out_hbm,
               idx_a, idx_b, rows_a, rows_b, stage_a, stage_b, sem_a, sem_b):
    wid = lax.axis_index("s") * NC + lax.axis_index("c")
    c0 = wid * CPW

    idx_bufs = (idx_a, idx_b)
    row_bufs = (rows_a, rows_b)
    stages = (stage_a, stage_b)
    sems = (sem_a, sem_b)

    def start(chunk, slot):
        pltpu.sync_copy(idx_hbm.at[pl.ds(chunk * IPC, IPC)], idx_bufs[slot])
        for j in range(NG):
            pltpu.async_copy(
                table_hbm.at[idx_bufs[slot].at[pl.ds(j * GSZ, GSZ)]],
                row_bufs[slot].at[pl.ds(j * GSZ, GSZ)],
                sems[slot],
            )

    def wait(slot):
        pltpu.make_async_copy(
            out_hbm.at[pl.ds(0, IPC)], row_bufs[slot], sems[slot]).wait()

    def compute(chunk, slot):
        rows = row_bufs[slot]
        stage = stages[slot]
        for k in range(CB):
            base = k * L

            def body(t, accs, base=base):
                a, b = accs
                r = base + t * 2
                a = tuple(a[j] + rows[r, pl.ds(j * 16, 16)] for j in range(4))
                b = tuple(b[j] + rows[r + 1, pl.ds(j * 16, 16)]
                          for j in range(4))
                return (a, b)

            z4 = tuple(jnp.zeros((16,), jnp.float32) for _ in range(4))
            a, b = lax.fori_loop(0, L // 2, body, (z4, z4))
            for j in range(4):
                stage[k, pl.ds(j * 16, 16)] = (a[j] + b[j]) * (1.0 / L)
        pltpu.sync_copy(stage, out_hbm.at[pl.ds(chunk * CB, CB)])

    start(c0 + 0, 0)
    start(c0 + 1, 1)

    def loop_body(c):
        for b in range(2):
            wait(b)
            compute(c0 + c + b, b)
            start(c0 + c + b + 2, b)

    pl.loop(0, CPW - 2, step=2)(loop_body)

    for b in range(2):
        wait(b)
        compute(c0 + CPW - 2 + b, b)


@functools.cache
def _make_pool():
    return pl.kernel(
        _pool_body,
        out_type=jax.ShapeDtypeStruct((B, EMB), jnp.float32),
        mesh=plsc.VectorSubcoreMesh(core_axis_name="c", subcore_axis_name="s",
                                    num_cores=NC, num_subcores=NS),
        scratch_types=[
            pltpu.VMEM((IPC,), jnp.int32),
            pltpu.VMEM((IPC,), jnp.int32),
            pltpu.VMEM((IPC, EMB), jnp.float32),
            pltpu.VMEM((IPC, EMB), jnp.float32),
            pltpu.VMEM((CB, EMB), jnp.float32),
            pltpu.VMEM((CB, EMB), jnp.float32),
            pltpu.SemaphoreType.DMA,
            pltpu.SemaphoreType.DMA,
        ],
        compiler_params=pltpu.CompilerParams(use_tc_tiling_on_sc=False),
    )


TBN = 8192


def _tr_body(x_ref, o_ref):
    o_ref[:, 0:EMB] = x_ref[...].T


_transpose = pl.pallas_call(
    _tr_body,
    grid=(pl.cdiv(VOCAB, TBN),),
    in_specs=[pl.BlockSpec((EMB, TBN), lambda i: (0, i))],
    out_specs=pl.BlockSpec((TBN, 2 * EMB), lambda i: (i, 0)),
    out_shape=jax.ShapeDtypeStruct((VOCAB, 2 * EMB), jnp.float32),
)


def _mlp_body(x_ref, w1_ref, b1_ref, w2_ref, b2_ref, o_ref):
    x = x_ref[...]
    h = lax.dot_general(w1_ref[...], x, (((0,), (1,)), ((), ())),
                        preferred_element_type=jnp.float32)
    h = jnp.maximum(h + b1_ref[...], 0.0)
    logits = lax.dot_general(w2_ref[...], h, (((0,), (0,)), ((), ())),
                             preferred_element_type=jnp.float32)
    logits = logits + b2_ref[...]
    m = jnp.max(logits, axis=0, keepdims=True)
    ex = jnp.exp(logits - m)
    s = jnp.sum(ex, axis=0, keepdims=True)
    o_ref[...] = (logits - m - jnp.log(s))[:NCLS, :]


_mlp = pl.pallas_call(
    _mlp_body,
    grid=(B // BM,),
    in_specs=[
        pl.BlockSpec((BM, EMB), lambda i: (i, 0)),
        pl.BlockSpec((EMB, HIDP), lambda i: (0, 0)),
        pl.BlockSpec((HIDP, 1), lambda i: (0, 0)),
        pl.BlockSpec((HIDP, NCLSP), lambda i: (0, 0)),
        pl.BlockSpec((NCLSP, 1), lambda i: (0, 0)),
    ],
    out_specs=pl.BlockSpec((NCLS, BM), lambda i: (0, i)),
    out_shape=jax.ShapeDtypeStruct((NCLS, B), jnp.float32),
)


def kernel(indices, emb_table, W1, b1, W2, b2):
    idx_flat = indices.reshape(B * L) * 2
    table_lin = _transpose(emb_table.T).reshape(2 * VOCAB, EMB)
    pooled = _make_pool()(idx_flat, table_lin)
    W1p = jnp.pad(W1, ((0, 0), (0, HIDP - HID)))
    b1p = jnp.pad(b1, (0, HIDP - HID)).reshape(HIDP, 1)
    W2p = jnp.pad(W2, ((0, HIDP - HID), (0, NCLSP - NCLS)))
    b2p = jnp.pad(b2, (0, NCLSP - NCLS),
                  constant_values=-1e9).reshape(NCLSP, 1)
    out_t = _mlp(pooled, W1p, b1p, W2p, b2p)
    return out_t.T


# speedup vs baseline: 1.0386x; 1.0386x over previous
import functools

import jax
import jax.numpy as jnp
from jax import lax
from jax.experimental import pallas as pl
from jax.experimental.pallas import tpu as pltpu
from jax.experimental.pallas import tpu_sc as plsc

VOCAB = 1000000
EMB = 64
HID = 1000
NCLS = 1000
B = 16384
L = 200

NC, NS = 2, 16
NW = NC * NS
CB = 4
GSZ = 80
IPC = CB * L
NG = IPC // GSZ
NCHUNK = B // CB
CPW = NCHUNK // NW

HIDP = 1024
NCLSP = 1024
BM = 1024


def _pool_body(idx_hbm, table_hbm, out_hbm,
               idx_a, idx_b, rows_a, rows_b, stage_a, stage_b, sem_a, sem_b):
    wid = lax.axis_index("s") * NC + lax.axis_index("c")
    c0 = wid * CPW

    idx_bufs = (idx_a, idx_b)
    row_bufs = (rows_a, rows_b)
    stages = (stage_a, stage_b)
    sems = (sem_a, sem_b)

    def start(chunk, slot):
        pltpu.sync_copy(idx_hbm.at[pl.ds(chunk * IPC, IPC)], idx_bufs[slot])
        for j in range(NG):
            pltpu.async_copy(
                table_hbm.at[idx_bufs[slot].at[pl.ds(j * GSZ, GSZ)]],
                row_bufs[slot].at[pl.ds(j * GSZ, GSZ)],
                sems[slot],
            )

    def wait(slot):
        pltpu.make_async_copy(
            out_hbm.at[pl.ds(0, IPC)], row_bufs[slot], sems[slot]).wait()

    def compute(chunk, slot):
        rows = row_bufs[slot]
        stage = stages[slot]
        for k in range(CB):
            base = k * L

            def body(t, accs, base=base):
                a, b = accs
                r = base + t * 2
                a = tuple(a[j] + rows[r, pl.ds(j * 16, 16)] for j in range(4))
                b = tuple(b[j] + rows[r + 1, pl.ds(j * 16, 16)]
                          for j in range(4))
                return (a, b)

            z4 = tuple(jnp.zeros((16,), jnp.float32) for _ in range(4))
            a, b = lax.fori_loop(0, L // 2, body, (z4, z4))
            for j in range(4):
                stage[k, pl.ds(j * 16, 16)] = (a[j] + b[j]) * (1.0 / L)
        pltpu.sync_copy(stage, out_hbm.at[pl.ds(chunk * CB, CB)])

    start(c0 + 0, 0)
    start(c0 + 1, 1)

    def loop_body(c):
        for b in range(2):
            wait(b)
            compute(c0 + c + b, b)
            start(c0 + c + b + 2, b)

    pl.loop(0, CPW - 2, step=2)(loop_body)

    for b in range(2):
        wait(b)
        compute(c0 + CPW - 2 + b, b)


@functools.cache
def _make_pool():
    return pl.kernel(
        _pool_body,
        out_type=jax.ShapeDtypeStruct((B, EMB), jnp.float32),
        mesh=plsc.VectorSubcoreMesh(core_axis_name="c", subcore_axis_name="s",
                                    num_cores=NC, num_subcores=NS),
        scratch_types=[
            pltpu.VMEM((IPC,), jnp.int32),
            pltpu.VMEM((IPC,), jnp.int32),
            pltpu.VMEM((IPC, EMB), jnp.float32),
            pltpu.VMEM((IPC, EMB), jnp.float32),
            pltpu.VMEM((CB, EMB), jnp.float32),
            pltpu.VMEM((CB, EMB), jnp.float32),
            pltpu.SemaphoreType.DMA,
            pltpu.SemaphoreType.DMA,
        ],
        compiler_params=pltpu.CompilerParams(use_tc_tiling_on_sc=False),
    )


TBN = 32768


def _tr_body(x_ref, o_ref):
    o_ref[:, 0:EMB] = x_ref[...].T


_transpose = pl.pallas_call(
    _tr_body,
    grid=(pl.cdiv(VOCAB, TBN),),
    in_specs=[pl.BlockSpec((EMB, TBN), lambda i: (0, i))],
    out_specs=pl.BlockSpec((TBN, 2 * EMB), lambda i: (i, 0)),
    out_shape=jax.ShapeDtypeStruct((VOCAB, 2 * EMB), jnp.float32),
)


def _mlp_body(x_ref, w1_ref, b1_ref, w2_ref, b2_ref, o_ref):
    x = x_ref[...]
    h = lax.dot_general(w1_ref[...], x, (((0,), (1,)), ((), ())),
                        preferred_element_type=jnp.float32)
    h = jnp.maximum(h + b1_ref[...], 0.0)
    logits = lax.dot_general(w2_ref[...], h, (((0,), (0,)), ((), ())),
                             preferred_element_type=jnp.float32)
    logits = logits + b2_ref[...]
    m = jnp.max(logits, axis=0, keepdims=True)
    ex = jnp.exp(logits - m)
    s = jnp.sum(ex, axis=0, keepdims=True)
    o_ref[...] = (logits - m - jnp.log(s))[:NCLS, :]


_mlp = pl.pallas_call(
    _mlp_body,
    grid=(B // BM,),
    in_specs=[
        pl.BlockSpec((BM, EMB), lambda i: (i, 0)),
        pl.BlockSpec((EMB, HIDP), lambda i: (0, 0)),
        pl.BlockSpec((HIDP, 1), lambda i: (0, 0)),
        pl.BlockSpec((HIDP, NCLSP), lambda i: (0, 0)),
        pl.BlockSpec((NCLSP, 1), lambda i: (0, 0)),
    ],
    out_specs=pl.BlockSpec((NCLS, BM), lambda i: (0, i)),
    out_shape=jax.ShapeDtypeStruct((NCLS, B), jnp.float32),
)


def kernel(indices, emb_table, W1, b1, W2, b2):
    idx_flat = indices.reshape(B * L) * 2
    table_lin = _transpose(emb_table.T).reshape(2 * VOCAB, EMB)
    pooled = _make_pool()(idx_flat, table_lin)
    W1p = jnp.pad(W1, ((0, 0), (0, HIDP - HID)))
    b1p = jnp.pad(b1, (0, HIDP - HID)).reshape(HIDP, 1)
    W2p = jnp.pad(W2, ((0, HIDP - HID), (0, NCLSP - NCLS)))
    b2p = jnp.pad(b2, (0, NCLSP - NCLS),
                  constant_values=-1e9).reshape(NCLSP, 1)
    out_t = _mlp(pooled, W1p, b1p, W2p, b2p)
    return out_t.T


# speedup vs baseline: 1.0550x; 1.0157x over previous
import functools

import jax
import jax.numpy as jnp
from jax import lax
from jax.experimental import pallas as pl
from jax.experimental.pallas import tpu as pltpu
from jax.experimental.pallas import tpu_sc as plsc

VOCAB = 1000000
EMB = 64
HID = 1000
NCLS = 1000
B = 16384
L = 200

NC, NS = 2, 16
NW = NC * NS
CB = 4
GSZ = 80
IPC = CB * L
NG = IPC // GSZ
NCHUNK = B // CB
CPW = NCHUNK // NW

HIDP = 1024
NCLSP = 1024
BM = 1024


def _pool_body(cpw, idx_hbm, table_hbm, out_hbm,
               idx_a, idx_b, rows_a, rows_b, stage_a, stage_b, sem_a, sem_b):
    wid = lax.axis_index("s") * NC + lax.axis_index("c")
    c0 = wid * cpw

    idx_bufs = (idx_a, idx_b)
    row_bufs = (rows_a, rows_b)
    stages = (stage_a, stage_b)
    sems = (sem_a, sem_b)

    def start(chunk, slot):
        pltpu.sync_copy(idx_hbm.at[pl.ds(chunk * IPC, IPC)], idx_bufs[slot])
        for j in range(NG):
            pltpu.async_copy(
                table_hbm.at[idx_bufs[slot].at[pl.ds(j * GSZ, GSZ)]],
                row_bufs[slot].at[pl.ds(j * GSZ, GSZ)],
                sems[slot],
            )

    def wait(slot):
        pltpu.make_async_copy(
            out_hbm.at[pl.ds(0, IPC)], row_bufs[slot], sems[slot]).wait()

    def compute(chunk, slot):
        rows = row_bufs[slot]
        stage = stages[slot]
        for k in range(CB):
            base = k * L

            def body(t, accs, base=base):
                a, b = accs
                r = base + t * 2
                a = tuple(a[j] + rows[r, pl.ds(j * 16, 16)] for j in range(4))
                b = tuple(b[j] + rows[r + 1, pl.ds(j * 16, 16)]
                          for j in range(4))
                return (a, b)

            z4 = tuple(jnp.zeros((16,), jnp.float32) for _ in range(4))
            a, b = lax.fori_loop(0, L // 2, body, (z4, z4))
            for j in range(4):
                stage[k, pl.ds(j * 16, 16)] = (a[j] + b[j]) * (1.0 / L)
        pltpu.sync_copy(stage, out_hbm.at[pl.ds(chunk * CB, CB)])

    start(c0 + 0, 0)
    start(c0 + 1, 1)

    def loop_body(c):
        for b in range(2):
            wait(b)
            compute(c0 + c + b, b)
            start(c0 + c + b + 2, b)

    pl.loop(0, cpw - 2, step=2)(loop_body)

    for b in range(2):
        wait(b)
        compute(c0 + cpw - 2 + b, b)


@functools.cache
def _make_pool(nbags):
    cpw = (nbags // CB) // NW
    return pl.kernel(
        functools.partial(_pool_body, cpw),
        out_type=jax.ShapeDtypeStruct((nbags, EMB), jnp.float32),
        mesh=plsc.VectorSubcoreMesh(core_axis_name="c", subcore_axis_name="s",
                                    num_cores=NC, num_subcores=NS),
        scratch_types=[
            pltpu.VMEM((IPC,), jnp.int32),
            pltpu.VMEM((IPC,), jnp.int32),
            pltpu.VMEM((IPC, EMB), jnp.float32),
            pltpu.VMEM((IPC, EMB), jnp.float32),
            pltpu.VMEM((CB, EMB), jnp.float32),
            pltpu.VMEM((CB, EMB), jnp.float32),
            pltpu.SemaphoreType.DMA,
            pltpu.SemaphoreType.DMA,
        ],
        compiler_params=pltpu.CompilerParams(use_tc_tiling_on_sc=False),
    )


TBN = 32768


def _tr_body(x_ref, o_ref):
    o_ref[:, 0:EMB] = x_ref[...].T


_transpose = pl.pallas_call(
    _tr_body,
    grid=(pl.cdiv(VOCAB, TBN),),
    in_specs=[pl.BlockSpec((EMB, TBN), lambda i: (0, i))],
    out_specs=pl.BlockSpec((TBN, 2 * EMB), lambda i: (i, 0)),
    out_shape=jax.ShapeDtypeStruct((VOCAB, 2 * EMB), jnp.float32),
)


def _mlp_body(x_ref, w1_ref, b1_ref, w2_ref, b2_ref, o_ref):
    x = x_ref[...]
    h = lax.dot_general(w1_ref[...], x, (((0,), (1,)), ((), ())),
                        preferred_element_type=jnp.float32)
    h = jnp.maximum(h + b1_ref[...], 0.0)
    logits = lax.dot_general(w2_ref[...], h, (((0,), (0,)), ((), ())),
                             preferred_element_type=jnp.float32)
    logits = logits + b2_ref[...]
    m = jnp.max(logits, axis=0, keepdims=True)
    ex = jnp.exp(logits - m)
    s = jnp.sum(ex, axis=0, keepdims=True)
    o_ref[...] = (logits - m - jnp.log(s))[:NCLS, :]


HB = B // 2
NBM = HB // BM

_WEIGHT_SPECS = [
    pl.BlockSpec((EMB, HIDP), lambda i: (0, 0)),
    pl.BlockSpec((HIDP, 1), lambda i: (0, 0)),
    pl.BlockSpec((HIDP, NCLSP), lambda i: (0, 0)),
    pl.BlockSpec((NCLSP, 1), lambda i: (0, 0)),
]

_mlp_lo = pl.pallas_call(
    _mlp_body,
    grid=(NBM,),
    in_specs=[pl.BlockSpec((BM, EMB), lambda i: (i, 0))] + _WEIGHT_SPECS,
    out_specs=pl.BlockSpec((NCLS, BM), lambda i: (0, i)),
    out_shape=jax.ShapeDtypeStruct((NCLS, B), jnp.float32),
)


def _mlp_body_hi(x_ref, w1_ref, b1_ref, w2_ref, b2_ref, prev_ref, o_ref):
    del prev_ref
    _mlp_body(x_ref, w1_ref, b1_ref, w2_ref, b2_ref, o_ref)


_mlp_hi = pl.pallas_call(
    _mlp_body_hi,
    grid=(NBM,),
    in_specs=[pl.BlockSpec((BM, EMB), lambda i: (i, 0))] + _WEIGHT_SPECS
    + [pl.BlockSpec(memory_space=pltpu.MemorySpace.HBM)],
    out_specs=pl.BlockSpec((NCLS, BM), lambda i: (0, i + NBM)),
    out_shape=jax.ShapeDtypeStruct((NCLS, B), jnp.float32),
    input_output_aliases={5: 0},
)


def kernel(indices, emb_table, W1, b1, W2, b2):
    idx_flat = indices.reshape(B * L) * 2
    table_lin = _transpose(emb_table.T).reshape(2 * VOCAB, EMB)
    pool = _make_pool(HB)
    pooled1 = pool(idx_flat[:HB * L], table_lin)
    pooled2 = pool(idx_flat[HB * L:], table_lin)
    W1p = jnp.pad(W1, ((0, 0), (0, HIDP - HID)))
    b1p = jnp.pad(b1, (0, HIDP - HID)).reshape(HIDP, 1)
    W2p = jnp.pad(W2, ((0, HIDP - HID), (0, NCLSP - NCLS)))
    b2p = jnp.pad(b2, (0, NCLSP - NCLS),
                  constant_values=-1e9).reshape(NCLSP, 1)
    out1 = _mlp_lo(pooled1, W1p, b1p, W2p, b2p)
    out_t = _mlp_hi(pooled2, W1p, b1p, W2p, b2p, out1)
    return out_t.T
